# Initial kernel scaffold; baseline (speedup 1.0000x reference)
import functools

import jax
import jax.numpy as jnp
from jax import lax
from jax.experimental import pallas as pl
from jax.experimental.pallas import tpu as pltpu
from jax.experimental.pallas import tpu_sc as plsc

_N_FINE = 40000
_N_COARSE = 10000
_D = 128
_NC = 2
_NS = 16
_NW = _NC * _NS
_K = 128
_ACC = 10240
_STRIPE = _ACC // _NS
_LANES = 16


def _prep(src, dst, w, table_rows, npc):
    e = src.shape[0]
    tot = _NW * npc * _K
    pad = tot - e
    src = src.astype(jnp.int32)
    dst = dst.astype(jnp.int32)
    if pad:
        ar = jnp.arange(pad, dtype=jnp.int32)
        src = jnp.concatenate([src, ar % table_rows])
        dst = jnp.concatenate([dst, ar % _N_COARSE])
        w = jnp.concatenate([w, jnp.zeros((pad,), jnp.float32)])
    return (src.reshape(_NW, npc, _K), dst.reshape(_NW, npc, _K),
            w.reshape(_NW, npc, _K))


def _sc_pass(table, src3, dst3, w3):
    npc = src3.shape[1]
    mesh = plsc.VectorSubcoreMesh(core_axis_name="c", subcore_axis_name="s")

    @functools.partial(
        pl.kernel,
        out_type=jax.ShapeDtypeStruct((_NC, _ACC, _D), jnp.float32),
        mesh=mesh,
        scratch_types=[
            pltpu.VMEM((npc, _K), jnp.int32),
            pltpu.VMEM((npc, _K), jnp.int32),
            pltpu.VMEM((npc, _K), jnp.float32),
            pltpu.VMEM((_K, _D), jnp.float32),
            pltpu.VMEM_SHARED((_ACC, _D), jnp.float32),
        ],
    )
    def run(table_ref, src_ref, dst_ref, w_ref, out_ref,
            srcv, dstv, wv, rows, acc):
        cid = lax.axis_index("c")
        sid = lax.axis_index("s")
        wid = cid * _NS + sid

        pltpu.sync_copy(src_ref.at[wid], srcv)
        pltpu.sync_copy(dst_ref.at[wid], dstv)
        pltpu.sync_copy(w_ref.at[wid], wv)

        @pl.loop(0, _K)
        def _(e):
            for c in range(_D // _LANES):
                rows[e, pl.ds(c * _LANES, _LANES)] = jnp.zeros(
                    (_LANES,), jnp.float32)

        base = sid * _STRIPE
        for i in range(_STRIPE // _K):
            pltpu.sync_copy(rows, acc.at[pl.ds(base + i * _K, _K)])
        plsc.subcore_barrier()

        @pl.loop(0, npc)
        def _(j):
            pltpu.sync_copy(table_ref.at[srcv.at[j]], rows)

            @pl.loop(0, _K)
            def _(e):
                ws = wv[j, e]
                for c in range(_D // _LANES):
                    sl = pl.ds(c * _LANES, _LANES)
                    rows[e, sl] = rows[e, sl] * ws

            pltpu.sync_copy(rows, acc.at[dstv.at[j]], add=True)

        plsc.subcore_barrier()
        for i in range(_STRIPE // _K):
            sl = pl.ds(base + i * _K, _K)
            pltpu.sync_copy(acc.at[sl], out_ref.at[cid, sl])

    return run(table, src3, dst3, w3)


_BLK = 1000


def _tc_add(parts):
    def body(p_ref, o_ref):
        o_ref[...] = p_ref[0] + p_ref[1]

    return pl.pallas_call(
        body,
        grid=(_N_COARSE // _BLK,),
        in_specs=[pl.BlockSpec((2, _BLK, _D), lambda i: (0, i, 0))],
        out_specs=pl.BlockSpec((_BLK, _D), lambda i: (i, 0)),
        out_shape=jax.ShapeDtypeStruct((_N_COARSE, _D), jnp.float32),
    )(parts)


def _tc_layer(parts, w, b, relu):
    def body(p_ref, w_ref, b_ref, o_ref):
        s = p_ref[0] + p_ref[1]
        y = jnp.dot(s, w_ref[...], preferred_element_type=jnp.float32)
        y = y + b_ref[...]
        if relu:
            y = jnp.maximum(y, 0.0)
        o_ref[...] = y

    return pl.pallas_call(
        body,
        grid=(_N_COARSE // _BLK,),
        in_specs=[pl.BlockSpec((2, _BLK, _D), lambda i: (0, i, 0)),
                  pl.BlockSpec((_D, _D), lambda i: (0, 0)),
                  pl.BlockSpec((1, _D), lambda i: (0, 0))],
        out_specs=pl.BlockSpec((_BLK, _D), lambda i: (i, 0)),
        out_shape=jax.ShapeDtypeStruct((_N_COARSE, _D), jnp.float32),
    )(parts, w, b.reshape(1, _D))


def kernel(x, pool_src, pool_dst, pool_edge_attr, pp_edge_index, pp_edge_attr,
           W1, b1, W2, b2):
    ps, pd, pw = _prep(pool_src, pool_dst, pool_edge_attr, _N_FINE, 10)
    es, ed, ew = _prep(pp_edge_index[0], pp_edge_index[1], pp_edge_attr,
                       _N_COARSE, 80)

    pool_parts = _sc_pass(x, ps, pd, pw)
    h = _tc_add(pool_parts)
    parts1 = _sc_pass(h, es, ed, ew)
    h1 = _tc_layer(parts1, W1, b1, relu=True)
    parts2 = _sc_pass(h1, es, ed, ew)
    return _tc_layer(parts2, W2, b2, relu=False)


# speedup vs baseline: 6.1426x; 6.1426x over previous
import functools

import jax
import jax.numpy as jnp
from jax import lax
from jax.experimental import pallas as pl
from jax.experimental.pallas import tpu as pltpu
from jax.experimental.pallas import tpu_sc as plsc

_N_FINE = 40000
_N_COARSE = 10000
_D = 128
_NC = 2
_NS = 16
_NW = _NC * _NS
_K = 128
_ACC = 10240
_STRIPE = _ACC // _NS
_LANES = 16


def _prep(src, dst, w, table_rows, npc):
    e = src.shape[0]
    tot = _NW * npc * _K
    pad = tot - e
    src = src.astype(jnp.int32)
    dst = dst.astype(jnp.int32)
    if pad:
        ar = jnp.arange(pad, dtype=jnp.int32)
        src = jnp.concatenate([src, ar % table_rows])
        dst = jnp.concatenate([dst, ar % _N_COARSE])
        w = jnp.concatenate([w, jnp.zeros((pad,), jnp.float32)])
    return (src.reshape(_NW, npc, _K), dst.reshape(_NW, npc, _K),
            w.reshape(_NW, npc, _K))


def _sc_pass(table, src3, dst3, w3):
    npc = src3.shape[1]
    mesh = plsc.VectorSubcoreMesh(core_axis_name="c", subcore_axis_name="s")

    @functools.partial(
        pl.kernel,
        out_type=jax.ShapeDtypeStruct((_NC, _ACC, _D), jnp.float32),
        mesh=mesh,
        scratch_types=[
            pltpu.VMEM((npc, _K), jnp.int32),
            pltpu.VMEM((npc, _K), jnp.int32),
            pltpu.VMEM((npc, _K), jnp.float32),
            pltpu.VMEM((_K, _D), jnp.float32),
            pltpu.VMEM_SHARED((_ACC, _D), jnp.float32),
        ],
    )
    def run(table_ref, src_ref, dst_ref, w_ref, out_ref,
            srcv, dstv, wv, rows, acc):
        cid = lax.axis_index("c")
        sid = lax.axis_index("s")
        wid = cid * _NS + sid

        pltpu.sync_copy(src_ref.at[wid], srcv)
        pltpu.sync_copy(dst_ref.at[wid], dstv)
        pltpu.sync_copy(w_ref.at[wid], wv)

        @pl.loop(0, _K)
        def _(e):
            for c in range(_D // _LANES):
                rows[e, pl.ds(c * _LANES, _LANES)] = jnp.zeros(
                    (_LANES,), jnp.float32)

        base = sid * _STRIPE
        for i in range(_STRIPE // _K):
            pltpu.sync_copy(rows, acc.at[pl.ds(base + i * _K, _K)])
        plsc.subcore_barrier()

        @pl.loop(0, npc)
        def _(j):
            pltpu.sync_copy(table_ref.at[srcv.at[j]], rows)

            @pl.loop(0, _K // _LANES)
            def _(g):
                wvec = wv[j, pl.ds(g * _LANES, _LANES)]
                for t in range(_LANES):
                    ws = wvec[t]
                    e = g * _LANES + t
                    for c in range(_D // _LANES):
                        sl = pl.ds(c * _LANES, _LANES)
                        rows[e, sl] = rows[e, sl] * ws

            pltpu.sync_copy(rows, acc.at[dstv.at[j]], add=True)

        plsc.subcore_barrier()
        for i in range(_STRIPE // _K):
            sl = pl.ds(base + i * _K, _K)
            pltpu.sync_copy(acc.at[sl], out_ref.at[cid, sl])

    return run(table, src3, dst3, w3)


_BLK = 1000


def _tc_add(parts):
    def body(p_ref, o_ref):
        o_ref[...] = p_ref[0] + p_ref[1]

    return pl.pallas_call(
        body,
        grid=(_N_COARSE // _BLK,),
        in_specs=[pl.BlockSpec((2, _BLK, _D), lambda i: (0, i, 0))],
        out_specs=pl.BlockSpec((_BLK, _D), lambda i: (i, 0)),
        out_shape=jax.ShapeDtypeStruct((_N_COARSE, _D), jnp.float32),
    )(parts)


def _tc_layer(parts, w, b, relu):
    def body(p_ref, w_ref, b_ref, o_ref):
        s = p_ref[0] + p_ref[1]
        y = jnp.dot(s, w_ref[...], preferred_element_type=jnp.float32)
        y = y + b_ref[...]
        if relu:
            y = jnp.maximum(y, 0.0)
        o_ref[...] = y

    return pl.pallas_call(
        body,
        grid=(_N_COARSE // _BLK,),
        in_specs=[pl.BlockSpec((2, _BLK, _D), lambda i: (0, i, 0)),
                  pl.BlockSpec((_D, _D), lambda i: (0, 0)),
                  pl.BlockSpec((1, _D), lambda i: (0, 0))],
        out_specs=pl.BlockSpec((_BLK, _D), lambda i: (i, 0)),
        out_shape=jax.ShapeDtypeStruct((_N_COARSE, _D), jnp.float32),
    )(parts, w, b.reshape(1, _D))


def kernel(x, pool_src, pool_dst, pool_edge_attr, pp_edge_index, pp_edge_attr,
           W1, b1, W2, b2):
    ps, pd, pw = _prep(pool_src, pool_dst, pool_edge_attr, _N_FINE, 10)
    es, ed, ew = _prep(pp_edge_index[0], pp_edge_index[1], pp_edge_attr,
                       _N_COARSE, 80)

    pool_parts = _sc_pass(x, ps, pd, pw)
    h = _tc_add(pool_parts)
    parts1 = _sc_pass(h, es, ed, ew)
    h1 = _tc_layer(parts1, W1, b1, relu=True)
    parts2 = _sc_pass(h1, es, ed, ew)
    return _tc_layer(parts2, W2, b2, relu=False)


# speedup vs baseline: 9.6396x; 1.5693x over previous
import dataclasses
import functools

import jax
import jax.numpy as jnp
from jax import lax
from jax.experimental import pallas as pl
from jax.experimental.pallas import tpu as pltpu
from jax.experimental.pallas import tpu_sc as plsc

_N_FINE = 40000
_N_COARSE = 10000
_D = 128
_NC = 2
_NS = 16
_NW = _NC * _NS
_K = 128
_ACC = 10240
_STRIPE = _ACC // _NS
_LANES = 16


def _prep(src, dst, w, table_rows, npc):
    e = src.shape[0]
    tot = _NW * npc * _K
    pad = tot - e
    src = src.astype(jnp.int32)
    dst = dst.astype(jnp.int32)
    if pad:
        ar = jnp.arange(pad, dtype=jnp.int32)
        src = jnp.concatenate([src, ar % table_rows])
        dst = jnp.concatenate([dst, ar % _N_COARSE])
        w = jnp.concatenate([w, jnp.zeros((pad,), jnp.float32)])
    idx2 = jnp.stack([src.reshape(_NW, npc, _K),
                      dst.reshape(_NW, npc, _K)], axis=2)
    return idx2, w.reshape(_NW, npc, _K)


_NIDX = 4
_NROW = 2


def _sc_pass(table, idx2, w3):
    npc = idx2.shape[1]
    assert npc % _NIDX == 0 and npc >= _NIDX
    mesh = plsc.VectorSubcoreMesh(core_axis_name="c", subcore_axis_name="s")
    @functools.partial(
        pl.kernel,
        out_type=jax.ShapeDtypeStruct((_NC, _ACC, _D), jnp.float32),
        mesh=mesh,
        scratch_types=[
            pltpu.VMEM((_NIDX, 2, _K), jnp.int32),
            pltpu.VMEM((_NIDX, _K), jnp.float32),
            pltpu.VMEM((_NROW, _K, _D), jnp.float32),
            pltpu.VMEM_SHARED((_ACC, _D), jnp.float32),
            pltpu.SemaphoreType.DMA((_NIDX,)),
            pltpu.SemaphoreType.DMA((_NROW,)),
            pltpu.SemaphoreType.DMA((_NROW,)),
            pltpu.SemaphoreType.DMA,
        ],
    )
    def run(table_ref, e_ref, w_ref, out_ref, idx, wbuf, rows, acc, isem,
            gsem, ssem, dsem):
        cid = lax.axis_index("c")
        sid = lax.axis_index("s")
        wid = cid * _NS + sid

        @pl.loop(0, _K)
        def _(e):
            for c in range(_D // _LANES):
                rows[0, e, pl.ds(c * _LANES, _LANES)] = jnp.zeros(
                    (_LANES,), jnp.float32)

        base = sid * _STRIPE
        for i in range(_STRIPE // _K):
            pltpu.async_copy(rows.at[0], acc.at[pl.ds(base + i * _K, _K)],
                             dsem)
        for i in range(_STRIPE // _K):
            pltpu.make_async_copy(rows.at[0],
                                  acc.at[pl.ds(base + i * _K, _K)],
                                  dsem).wait()
        plsc.subcore_barrier()

        for j0 in range(_NIDX - 1):
            pltpu.async_copy(e_ref.at[wid, j0], idx.at[j0], isem.at[j0])
            pltpu.async_copy(w_ref.at[wid, j0], wbuf.at[j0], isem.at[j0])
        pltpu.make_async_copy(e_ref.at[wid, 0], idx.at[0], isem.at[0]).wait()
        pltpu.make_async_copy(w_ref.at[wid, 0], wbuf.at[0], isem.at[0]).wait()
        pltpu.async_copy(table_ref.at[idx.at[0, 0]], rows.at[0], gsem.at[0])

        @pl.loop(0, npc // _NIDX)
        def _(i):
            for p in range(_NIDX):
                j = i * _NIDX + p
                b = p % _NROW
                bn = (p + 1) % _NROW
                s = p
                sn = (p + 1) % _NIDX
                sl3 = (p + 3) % _NIDX

                def _drain():
                    pltpu.make_async_copy(
                        rows.at[bn], acc.at[idx.at[(s + 3) % _NIDX, 1]],
                        ssem.at[bn]).wait()
                if p == 0:
                    pl.when(j >= 1)(_drain)
                else:
                    _drain()

                @pl.when(j + 3 < npc)
                def _():
                    pltpu.async_copy(e_ref.at[wid, j + 3], idx.at[sl3],
                                     isem.at[sl3])
                    pltpu.async_copy(w_ref.at[wid, j + 3], wbuf.at[sl3],
                                     isem.at[sl3])

                @pl.when(j + 1 < npc)
                def _():
                    pltpu.make_async_copy(e_ref.at[wid, j + 1], idx.at[sn],
                                          isem.at[sn]).wait()
                    pltpu.make_async_copy(w_ref.at[wid, j + 1], wbuf.at[sn],
                                          isem.at[sn]).wait()
                    pltpu.async_copy(table_ref.at[idx.at[sn, 0]],
                                     rows.at[bn], gsem.at[bn])

                pltpu.make_async_copy(table_ref.at[idx.at[s, 0]], rows.at[b],
                                      gsem.at[b]).wait()

                @pl.loop(0, _K // _LANES)
                def _(g):
                    wvec = wbuf[s, pl.ds(g * _LANES, _LANES)]
                    for t in range(_LANES):
                        ws = wvec[t]
                        e = g * _LANES + t
                        for c in range(_D // _LANES):
                            csl = pl.ds(c * _LANES, _LANES)
                            rows[b, e, csl] = rows[b, e, csl] * ws

                pltpu.async_copy(rows.at[b], acc.at[idx.at[s, 1]],
                                 ssem.at[b], add=True)

        lb = (npc - 1) % _NROW
        ls = (npc - 1) % _NIDX
        pltpu.make_async_copy(rows.at[lb], acc.at[idx.at[ls, 1]],
                              ssem.at[lb]).wait()

        plsc.subcore_barrier()
        for i in range(_STRIPE // _K):
            sl = pl.ds(base + i * _K, _K)
            pltpu.async_copy(acc.at[sl], out_ref.at[cid, sl], dsem)
        for i in range(_STRIPE // _K):
            sl = pl.ds(base + i * _K, _K)
            pltpu.make_async_copy(acc.at[sl], out_ref.at[cid, sl],
                                  dsem).wait()

    return run(table, idx2, w3)


_BLK = 1000


def _tc_add(parts):
    def body(p_ref, o_ref):
        o_ref[...] = p_ref[0] + p_ref[1]

    return pl.pallas_call(
        body,
        grid=(_N_COARSE // _BLK,),
        in_specs=[pl.BlockSpec((2, _BLK, _D), lambda i: (0, i, 0))],
        out_specs=pl.BlockSpec((_BLK, _D), lambda i: (i, 0)),
        out_shape=jax.ShapeDtypeStruct((_N_COARSE, _D), jnp.float32),
    )(parts)


def _tc_layer(parts, w, b, relu):
    def body(p_ref, w_ref, b_ref, o_ref):
        s = p_ref[0] + p_ref[1]
        y = jnp.dot(s, w_ref[...], preferred_element_type=jnp.float32)
        y = y + b_ref[...]
        if relu:
            y = jnp.maximum(y, 0.0)
        o_ref[...] = y

    return pl.pallas_call(
        body,
        grid=(_N_COARSE // _BLK,),
        in_specs=[pl.BlockSpec((2, _BLK, _D), lambda i: (0, i, 0)),
                  pl.BlockSpec((_D, _D), lambda i: (0, 0)),
                  pl.BlockSpec((1, _D), lambda i: (0, 0))],
        out_specs=pl.BlockSpec((_BLK, _D), lambda i: (i, 0)),
        out_shape=jax.ShapeDtypeStruct((_N_COARSE, _D), jnp.float32),
    )(parts, w, b.reshape(1, _D))


def kernel(x, pool_src, pool_dst, pool_edge_attr, pp_edge_index, pp_edge_attr,
           W1, b1, W2, b2):
    pi, pw = _prep(pool_src, pool_dst, pool_edge_attr, _N_FINE, 12)
    ei, ew = _prep(pp_edge_index[0], pp_edge_index[1], pp_edge_attr,
                   _N_COARSE, 80)

    pool_parts = _sc_pass(x, pi, pw)
    h = _tc_add(pool_parts)
    parts1 = _sc_pass(h, ei, ew)
    h1 = _tc_layer(parts1, W1, b1, relu=True)
    parts2 = _sc_pass(h1, ei, ew)
    return _tc_layer(parts2, W2, b2, relu=False)
